# Initial kernel scaffold; baseline (speedup 1.0000x reference)
import jax
import jax.numpy as jnp
from jax.experimental import pallas as pl


def kernel(x):
    raise NotImplementedError("write your pallas kernel here")


# speedup vs baseline: 1.1286x; 1.1286x over previous
import jax
import jax.numpy as jnp
from jax import lax
from jax.experimental import pallas as pl
from jax.experimental.pallas import tpu as pltpu

_MASK_RATIO = 0.15
_ROW_BLOCK = 256


def _mask_mul_kernel(idx_ref, x_ref, o_ref, mask_ref):
    @pl.when(pl.program_id(0) == 0)
    def _build():
        idx = idx_ref[...]
        t_iota = lax.broadcasted_iota(jnp.int32, (idx.shape[0], mask_ref.shape[1]), 1)
        hit = jnp.any(idx == t_iota, axis=0, keepdims=True)
        mask_ref[...] = jnp.where(hit, 0.0, 1.0).astype(jnp.float32)

    o_ref[...] = x_ref[...] * mask_ref[...]


def kernel(x):
    B, C, T = x.shape
    n_mask = int(T * _MASK_RATIO)
    if n_mask <= 0:
        return x

    key = jax.random.fold_in(jax.random.key(0), 1)
    mask_indices = jax.random.permutation(key, T)[:n_mask].astype(jnp.int32)

    idx_pad = ((n_mask + 7) // 8) * 8
    idx2d = jnp.concatenate(
        [mask_indices, jnp.full((idx_pad - n_mask,), T, jnp.int32)]
    ).reshape(idx_pad, 1)

    rows = B * C
    xr = x.reshape(rows, T)
    grid = (rows // _ROW_BLOCK,)

    out = pl.pallas_call(
        _mask_mul_kernel,
        grid=grid,
        in_specs=[
            pl.BlockSpec((idx_pad, 1), lambda i: (0, 0)),
            pl.BlockSpec((_ROW_BLOCK, T), lambda i: (i, 0)),
        ],
        out_specs=pl.BlockSpec((_ROW_BLOCK, T), lambda i: (i, 0)),
        out_shape=jax.ShapeDtypeStruct((rows, T), x.dtype),
        scratch_shapes=[pltpu.VMEM((1, T), jnp.float32)],
    )(idx2d, xr)
    return out.reshape(B, C, T)
